# Initial kernel scaffold; baseline (speedup 1.0000x reference)
import jax
import jax.numpy as jnp
from jax.experimental import pallas as pl


def kernel(x, atten):
    raise NotImplementedError("write your pallas kernel here")


# speedup vs baseline: 1.8225x; 1.8225x over previous
import functools

import jax
import jax.numpy as jnp
from jax import lax
from jax.experimental import pallas as pl
from jax.experimental.pallas import tpu as pltpu

INDEX = 512
HEAD_NUM = 12
B = 2
S = 2048
D = 768
K_TOP = INDEX - 2
N_SEL = K_TOP + 1
N_OTHER = S - INDEX + 1

ROWS_BLK = 512


def _colsum_kernel(a_ref, o_ref):
    bh = pl.program_id(0)
    r = pl.program_id(1)
    blk = a_ref[0]
    row0 = r * ROWS_BLK
    i_idx = lax.broadcasted_iota(jnp.int32, (ROWS_BLK, S), 0) + row0
    j_idx = lax.broadcasted_iota(jnp.int32, (ROWS_BLK, S), 1)
    masked = jnp.where(i_idx == j_idx, 0.0, blk)
    contrib = jnp.sum(masked, axis=0, keepdims=True)

    @pl.when(jnp.logical_and(lax.rem(bh, HEAD_NUM) == 0, r == 0))
    def _():
        o_ref[...] = jnp.zeros_like(o_ref)

    o_ref[0] += contrib


def _select_kernel(a_ref, x_ref, o_ref):
    a = a_ref[0] * (1.0 / HEAD_NUM)
    jvec = lax.broadcasted_iota(jnp.int32, (1, S), 1)
    valid = jvec >= 1

    u = lax.bitcast_convert_type(a, jnp.uint32)
    key = jnp.where(
        (u >> 31) == 1, ~u, u | jnp.uint32(0x80000000)
    )
    key = jnp.where(valid, key, jnp.uint32(0))

    def body(k, t):
        cand = t | (jnp.uint32(1) << (jnp.uint32(31) - k.astype(jnp.uint32)))
        cnt = jnp.sum((key >= cand).astype(jnp.int32))
        return jnp.where(cnt >= K_TOP, cand, t)

    thr = lax.fori_loop(0, 32, body, jnp.uint32(0))

    lt = (
        lax.broadcasted_iota(jnp.int32, (S, S), 0)
        < lax.broadcasted_iota(jnp.int32, (S, S), 1)
    ).astype(jnp.float32)

    def prefix_excl(v):
        return lax.dot_general(
            v, lt, (((1,), (0,)), ((), ())),
            preferred_element_type=jnp.float32,
        )

    gt = jnp.logical_and(key > thr, valid)
    eq = jnp.logical_and(key == thr, valid)
    n_gt = jnp.sum(gt.astype(jnp.int32))
    need_eq = K_TOP - n_gt
    eq_f = eq.astype(jnp.float32)
    eq_pref = prefix_excl(eq_f)
    sel_rest = jnp.logical_or(gt, jnp.logical_and(eq, eq_pref < need_eq.astype(jnp.float32)))
    sel_full = jnp.logical_or(sel_rest, jvec == 0)

    sel_f = sel_full.astype(jnp.float32)
    pos = prefix_excl(sel_f)

    other = jnp.logical_and(valid, jnp.logical_not(sel_rest))
    neg_inf = jnp.float32(-jnp.inf)
    m = jnp.max(jnp.where(other, a, neg_inf))
    e = jnp.where(other, jnp.exp(a - m), 0.0)
    z = jnp.sum(e)
    w = e / (z * N_OTHER)

    prow = lax.broadcasted_iota(jnp.int32, (INDEX, S), 0)
    onehot = jnp.logical_and(prow == pos.astype(jnp.int32), sel_full)
    mat = onehot.astype(jnp.float32) + jnp.where(prow == INDEX - 1, w, 0.0)
    o_ref[0] = lax.dot_general(
        mat, x_ref[0],
        (((1,), (0,)), ((), ())),
        precision=lax.Precision.HIGHEST,
        preferred_element_type=jnp.float32,
    )


def _attended(atten):
    return pl.pallas_call(
        _colsum_kernel,
        grid=(B * HEAD_NUM, S // ROWS_BLK),
        in_specs=[pl.BlockSpec((1, ROWS_BLK, S), lambda bh, r: (bh, r, 0))],
        out_specs=pl.BlockSpec((1, 1, S), lambda bh, r: (bh // HEAD_NUM, 0, 0)),
        out_shape=jax.ShapeDtypeStruct((B, 1, S), jnp.float32),
        compiler_params=pltpu.CompilerParams(
            dimension_semantics=("arbitrary", "arbitrary"),
        ),
    )(atten)


def _extract(attended, x):
    return pl.pallas_call(
        _select_kernel,
        grid=(B,),
        in_specs=[
            pl.BlockSpec((1, 1, S), lambda b: (b, 0, 0)),
            pl.BlockSpec((1, S, D), lambda b: (b, 0, 0)),
        ],
        out_specs=pl.BlockSpec((1, INDEX, D), lambda b: (b, 0, 0)),
        out_shape=jax.ShapeDtypeStruct((B, INDEX, D), jnp.float32),
        compiler_params=pltpu.CompilerParams(
            dimension_semantics=("arbitrary",),
        ),
    )(attended, x)


@jax.jit
def kernel(x, atten):
    attended = _attended(atten)
    return _extract(attended, x)


# speedup vs baseline: 2.0627x; 1.1318x over previous
import functools

import jax
import jax.numpy as jnp
from jax import lax
from jax.experimental import pallas as pl
from jax.experimental.pallas import tpu as pltpu

INDEX = 512
HEAD_NUM = 12
B = 2
S = 2048
D = 768
K_TOP = INDEX - 2
N_SEL = K_TOP + 1
N_OTHER = S - INDEX + 1

ROWS_BLK = 1024


def _colsum_kernel(a_ref, o_ref):
    bh = pl.program_id(0)
    r = pl.program_id(1)
    blk = a_ref[0]
    row0 = r * ROWS_BLK
    i_idx = lax.broadcasted_iota(jnp.int32, (ROWS_BLK, S), 0) + row0
    j_idx = lax.broadcasted_iota(jnp.int32, (ROWS_BLK, S), 1)
    masked = jnp.where(i_idx == j_idx, 0.0, blk)
    contrib = jnp.sum(masked, axis=0, keepdims=True)

    @pl.when(jnp.logical_and(lax.rem(bh, HEAD_NUM) == 0, r == 0))
    def _():
        o_ref[...] = jnp.zeros_like(o_ref)

    o_ref[0] += contrib


def _select_kernel(a_ref, x_ref, o_ref):
    a = a_ref[0] * (1.0 / HEAD_NUM)
    jvec = lax.broadcasted_iota(jnp.int32, (1, S), 1)
    valid = jvec >= 1

    u = lax.bitcast_convert_type(a, jnp.uint32)
    key = jnp.where(
        (u >> 31) == 1, ~u, u | jnp.uint32(0x80000000)
    )
    key = jnp.where(valid, key, jnp.uint32(0))

    def body(k, t):
        cand = t | (jnp.uint32(1) << (jnp.uint32(31) - k.astype(jnp.uint32)))
        cnt = jnp.sum((key >= cand).astype(jnp.int32))
        return jnp.where(cnt >= K_TOP, cand, t)

    thr = lax.fori_loop(0, 32, body, jnp.uint32(0))

    def prefix_excl(v):
        acc = v
        for k in (1, 2, 4, 8, 16, 32, 64, 128, 256, 512, 1024):
            acc = acc + jnp.where(jvec >= k, pltpu.roll(acc, k, 1), 0.0)
        return acc - v

    gt = jnp.logical_and(key > thr, valid)
    eq = jnp.logical_and(key == thr, valid)
    n_gt = jnp.sum(gt.astype(jnp.int32))
    need_eq = K_TOP - n_gt
    eq_f = eq.astype(jnp.float32)
    eq_pref = prefix_excl(eq_f)
    sel_rest = jnp.logical_or(gt, jnp.logical_and(eq, eq_pref < need_eq.astype(jnp.float32)))
    sel_full = jnp.logical_or(sel_rest, jvec == 0)

    sel_f = sel_full.astype(jnp.float32)
    pos = prefix_excl(sel_f)

    other = jnp.logical_and(valid, jnp.logical_not(sel_rest))
    neg_inf = jnp.float32(-jnp.inf)
    m = jnp.max(jnp.where(other, a, neg_inf))
    e = jnp.where(other, jnp.exp(a - m), 0.0)
    z = jnp.sum(e)
    w = e / (z * N_OTHER)

    prow = lax.broadcasted_iota(jnp.int32, (INDEX, S), 0)
    onehot = jnp.logical_and(prow == pos.astype(jnp.int32), sel_full)
    mat = onehot.astype(jnp.float32) + jnp.where(prow == INDEX - 1, w, 0.0)
    o_ref[0] = lax.dot_general(
        mat, x_ref[0],
        (((1,), (0,)), ((), ())),
        precision=lax.Precision.HIGHEST,
        preferred_element_type=jnp.float32,
    )


def _attended(atten):
    return pl.pallas_call(
        _colsum_kernel,
        grid=(B * HEAD_NUM, S // ROWS_BLK),
        in_specs=[pl.BlockSpec((1, ROWS_BLK, S), lambda bh, r: (bh, r, 0))],
        out_specs=pl.BlockSpec((1, 1, S), lambda bh, r: (bh // HEAD_NUM, 0, 0)),
        out_shape=jax.ShapeDtypeStruct((B, 1, S), jnp.float32),
        compiler_params=pltpu.CompilerParams(
            dimension_semantics=("arbitrary", "arbitrary"),
        ),
    )(atten)


def _extract(attended, x):
    return pl.pallas_call(
        _select_kernel,
        grid=(B,),
        in_specs=[
            pl.BlockSpec((1, 1, S), lambda b: (b, 0, 0)),
            pl.BlockSpec((1, S, D), lambda b: (b, 0, 0)),
        ],
        out_specs=pl.BlockSpec((1, INDEX, D), lambda b: (b, 0, 0)),
        out_shape=jax.ShapeDtypeStruct((B, INDEX, D), jnp.float32),
        compiler_params=pltpu.CompilerParams(
            dimension_semantics=("arbitrary",),
        ),
    )(attended, x)


@jax.jit
def kernel(x, atten):
    attended = _attended(atten)
    return _extract(attended, x)


# speedup vs baseline: 2.1941x; 1.0637x over previous
import functools

import jax
import jax.numpy as jnp
from jax import lax
from jax.experimental import pallas as pl
from jax.experimental.pallas import tpu as pltpu

INDEX = 512
HEAD_NUM = 12
B = 2
S = 2048
D = 768
K_TOP = INDEX - 2
N_SEL = K_TOP + 1
N_OTHER = S - INDEX + 1

ROWS_BLK = 2048


def _colsum_kernel(a_ref, o_ref):
    bh = pl.program_id(0)
    r = pl.program_id(1)
    blk = a_ref[0]
    row0 = r * ROWS_BLK
    i_idx = lax.broadcasted_iota(jnp.int32, (ROWS_BLK, S), 0) + row0
    j_idx = lax.broadcasted_iota(jnp.int32, (ROWS_BLK, S), 1)
    masked = jnp.where(i_idx == j_idx, 0.0, blk)
    contrib = jnp.sum(masked, axis=0, keepdims=True)

    @pl.when(jnp.logical_and(lax.rem(bh, HEAD_NUM) == 0, r == 0))
    def _():
        o_ref[...] = jnp.zeros_like(o_ref)

    o_ref[0] += contrib


def _select_kernel(a_ref, x_ref, o_ref):
    a = a_ref[0] * (1.0 / HEAD_NUM)
    jvec = lax.broadcasted_iota(jnp.int32, (1, S), 1)
    valid = jvec >= 1

    u = lax.bitcast_convert_type(a, jnp.uint32)
    key = jnp.where(
        (u >> 31) == 1, ~u, u | jnp.uint32(0x80000000)
    )
    key = jnp.where(valid, key, jnp.uint32(0))

    def body(k, t):
        cand = t | (jnp.uint32(1) << (jnp.uint32(31) - k.astype(jnp.uint32)))
        cnt = jnp.sum((key >= cand).astype(jnp.int32))
        return jnp.where(cnt >= K_TOP, cand, t)

    thr = lax.fori_loop(0, 32, body, jnp.uint32(0))

    def prefix_excl(v):
        acc = v
        for k in (1, 2, 4, 8, 16, 32, 64, 128, 256, 512, 1024):
            acc = acc + jnp.where(jvec >= k, pltpu.roll(acc, k, 1), 0.0)
        return acc - v

    gt = jnp.logical_and(key > thr, valid)
    eq = jnp.logical_and(key == thr, valid)
    n_gt = jnp.sum(gt.astype(jnp.int32))
    need_eq = K_TOP - n_gt
    eq_f = eq.astype(jnp.float32)
    eq_pref = prefix_excl(eq_f)
    sel_rest = jnp.logical_or(gt, jnp.logical_and(eq, eq_pref < need_eq.astype(jnp.float32)))
    sel_full = jnp.logical_or(sel_rest, jvec == 0)

    sel_f = sel_full.astype(jnp.float32)
    pos = prefix_excl(sel_f)

    other = jnp.logical_and(valid, jnp.logical_not(sel_rest))
    neg_inf = jnp.float32(-jnp.inf)
    m = jnp.max(jnp.where(other, a, neg_inf))
    e = jnp.where(other, jnp.exp(a - m), 0.0)
    z = jnp.sum(e)
    w = e / (z * N_OTHER)

    prow = lax.broadcasted_iota(jnp.int32, (INDEX, S), 0)
    onehot = jnp.logical_and(prow == pos.astype(jnp.int32), sel_full)
    mat = onehot.astype(jnp.bfloat16)
    xv = x_ref[0]
    x_hi = xv.astype(jnp.bfloat16)
    x_lo = (xv - x_hi.astype(jnp.float32)).astype(jnp.bfloat16)
    dims = (((1,), (0,)), ((), ()))
    extract = lax.dot_general(
        mat, x_hi, dims, preferred_element_type=jnp.float32
    ) + lax.dot_general(mat, x_lo, dims, preferred_element_type=jnp.float32)
    wrow = lax.dot_general(
        w, xv, dims,
        precision=lax.Precision.HIGHEST,
        preferred_element_type=jnp.float32,
    )
    o_ref[0] = extract
    o_ref[0, INDEX - 1 : INDEX, :] = wrow


def _attended(atten):
    return pl.pallas_call(
        _colsum_kernel,
        grid=(B * HEAD_NUM, S // ROWS_BLK),
        in_specs=[pl.BlockSpec((1, ROWS_BLK, S), lambda bh, r: (bh, r, 0))],
        out_specs=pl.BlockSpec((1, 1, S), lambda bh, r: (bh // HEAD_NUM, 0, 0)),
        out_shape=jax.ShapeDtypeStruct((B, 1, S), jnp.float32),
        compiler_params=pltpu.CompilerParams(
            dimension_semantics=("arbitrary", "arbitrary"),
        ),
    )(atten)


def _extract(attended, x):
    return pl.pallas_call(
        _select_kernel,
        grid=(B,),
        in_specs=[
            pl.BlockSpec((1, 1, S), lambda b: (b, 0, 0)),
            pl.BlockSpec((1, S, D), lambda b: (b, 0, 0)),
        ],
        out_specs=pl.BlockSpec((1, INDEX, D), lambda b: (b, 0, 0)),
        out_shape=jax.ShapeDtypeStruct((B, INDEX, D), jnp.float32),
        compiler_params=pltpu.CompilerParams(
            dimension_semantics=("arbitrary",),
        ),
    )(attended, x)


@jax.jit
def kernel(x, atten):
    attended = _attended(atten)
    return _extract(attended, x)
